# baseline (speedup 1.0000x reference)
import numpy as np
import jax
import jax.numpy as jnp
from jax.experimental import pallas as pl
from jax.experimental.pallas import tpu as pltpu

_CLUB = np.array([[0, 1, 1, 1],
                  [1, 0, 0, 0],
                  [1, 0, 0, 1],
                  [1, 0, 1, 0]], dtype=np.float32)
_DEG = (_CLUB + np.eye(4, dtype=np.float32)).sum(axis=0)
_DINV = 1.0 / np.sqrt(_DEG)
_A = ((_CLUB + np.eye(4, dtype=np.float32)) * _DINV[None, :] * _DINV[:, None]).astype(np.float32)

_N = 100000
_T = 4000
_NB = 3


def _mix_first_rows(hb):
    rows = [hb[s:s + 1, :] for s in range(4)]
    mixed = jnp.concatenate(
        [sum(float(_A[d, s]) * rows[s] for s in range(4) if _A[d, s] != 0.0)
         for d in range(4)],
        axis=0,
    )
    return jnp.concatenate([mixed, hb[4:, :]], axis=0)


def _body(h_hbm, W_ref, b_ref, o_hbm, h_buf, o_buf, in_sem, out_sem):
    W = W_ref[...]
    bias = b_ref[...]
    num = _N // _T

    def in_copy(i):
        return pltpu.make_async_copy(
            h_hbm.at[pl.ds(i * _T, _T), :], h_buf.at[i % _NB], in_sem.at[i % _NB])

    def out_copy(i):
        return pltpu.make_async_copy(
            o_buf.at[i % _NB], o_hbm.at[pl.ds(i * _T, _T), :], out_sem.at[i % _NB])

    for k in range(min(_NB, num)):
        in_copy(k).start()
    for i in range(num):
        in_copy(i).wait()
        hb = h_buf[i % _NB]
        if i == 0:
            hb = _mix_first_rows(hb)
        x = jnp.dot(hb, W, preferred_element_type=jnp.float32)
        if i >= _NB:
            out_copy(i - _NB).wait()
        o_buf[i % _NB] = jnp.tanh(x + bias)
        out_copy(i).start()
        if i + _NB < num:
            in_copy(i + _NB).start()
    for i in range(max(0, num - _NB), num):
        out_copy(i).wait()


def kernel(t, h, W, b):
    del t
    N, D = h.shape
    out = pl.pallas_call(
        _body,
        in_specs=[
            pl.BlockSpec(memory_space=pl.ANY),
            pl.BlockSpec((D, D), lambda: (0, 0)),
            pl.BlockSpec((1, D), lambda: (0, 0)),
        ],
        out_specs=pl.BlockSpec(memory_space=pl.ANY),
        out_shape=jax.ShapeDtypeStruct((N, D), jnp.float32),
        scratch_shapes=[
            pltpu.VMEM((_NB, _T, D), jnp.float32),
            pltpu.VMEM((_NB, _T, D), jnp.float32),
            pltpu.SemaphoreType.DMA((_NB,)),
            pltpu.SemaphoreType.DMA((_NB,)),
        ],
    )(h, W, b.reshape(1, D))
    return out


# speedup vs baseline: 1.0229x; 1.0229x over previous
import numpy as np
import jax
import jax.numpy as jnp
from jax.experimental import pallas as pl
from jax.experimental.pallas import tpu as pltpu

_CLUB = np.array([[0, 1, 1, 1],
                  [1, 0, 0, 0],
                  [1, 0, 0, 1],
                  [1, 0, 1, 0]], dtype=np.float32)
_DEG = (_CLUB + np.eye(4, dtype=np.float32)).sum(axis=0)
_DINV = 1.0 / np.sqrt(_DEG)
_A = ((_CLUB + np.eye(4, dtype=np.float32)) * _DINV[None, :] * _DINV[:, None]).astype(np.float32)

_N = 100000
_T = 10000
_NB = 3


def _mix_first_rows(hb):
    rows = [hb[s:s + 1, :] for s in range(4)]
    mixed = jnp.concatenate(
        [sum(float(_A[d, s]) * rows[s] for s in range(4) if _A[d, s] != 0.0)
         for d in range(4)],
        axis=0,
    )
    return jnp.concatenate([mixed, hb[4:, :]], axis=0)


def _body(h_hbm, W_ref, b_ref, o_hbm, h_buf, o_buf, in_sem, out_sem):
    W = W_ref[...]
    bias = b_ref[...]
    num = _N // _T

    def in_copy(i):
        return pltpu.make_async_copy(
            h_hbm.at[pl.ds(i * _T, _T), :], h_buf.at[i % _NB], in_sem.at[i % _NB])

    def out_copy(i):
        return pltpu.make_async_copy(
            o_buf.at[i % _NB], o_hbm.at[pl.ds(i * _T, _T), :], out_sem.at[i % _NB])

    for k in range(min(_NB, num)):
        in_copy(k).start()
    for i in range(num):
        in_copy(i).wait()
        hb = h_buf[i % _NB]
        if i == 0:
            hb = _mix_first_rows(hb)
        x = jnp.dot(hb, W, preferred_element_type=jnp.float32)
        if i >= _NB:
            out_copy(i - _NB).wait()
        o_buf[i % _NB] = jnp.tanh(x + bias)
        out_copy(i).start()
        if i + _NB < num:
            in_copy(i + _NB).start()
    for i in range(max(0, num - _NB), num):
        out_copy(i).wait()


def kernel(t, h, W, b):
    del t
    N, D = h.shape
    out = pl.pallas_call(
        _body,
        in_specs=[
            pl.BlockSpec(memory_space=pl.ANY),
            pl.BlockSpec((D, D), lambda: (0, 0)),
            pl.BlockSpec((1, D), lambda: (0, 0)),
        ],
        out_specs=pl.BlockSpec(memory_space=pl.ANY),
        out_shape=jax.ShapeDtypeStruct((N, D), jnp.float32),
        scratch_shapes=[
            pltpu.VMEM((_NB, _T, D), jnp.float32),
            pltpu.VMEM((_NB, _T, D), jnp.float32),
            pltpu.SemaphoreType.DMA((_NB,)),
            pltpu.SemaphoreType.DMA((_NB,)),
        ],
    )(h, W, b.reshape(1, D))
    return out


# speedup vs baseline: 1.0578x; 1.0342x over previous
import numpy as np
import jax
import jax.numpy as jnp
from jax.experimental import pallas as pl
from jax.experimental.pallas import tpu as pltpu

_CLUB = np.array([[0, 1, 1, 1],
                  [1, 0, 0, 0],
                  [1, 0, 0, 1],
                  [1, 0, 1, 0]], dtype=np.float32)
_DEG = (_CLUB + np.eye(4, dtype=np.float32)).sum(axis=0)
_DINV = 1.0 / np.sqrt(_DEG)
_A = ((_CLUB + np.eye(4, dtype=np.float32)) * _DINV[None, :] * _DINV[:, None]).astype(np.float32)

_SIZES = [2000, 6000, 16000, 20000, 20000, 20000, 12000, 4000]
_OFFS = list(np.cumsum([0] + _SIZES[:-1]))
_TMAX = max(_SIZES)
_NB = 2


def _mix_first_rows(hb):
    rows = [hb[s:s + 1, :] for s in range(4)]
    mixed = jnp.concatenate(
        [sum(float(_A[d, s]) * rows[s] for s in range(4) if _A[d, s] != 0.0)
         for d in range(4)],
        axis=0,
    )
    return jnp.concatenate([mixed, hb[4:, :]], axis=0)


def _body(h_hbm, W_ref, b_ref, o_hbm, h_buf, o_buf, in_sem, out_sem):
    W = W_ref[...]
    bias = b_ref[...]
    num = len(_SIZES)

    def in_copy(i):
        return pltpu.make_async_copy(
            h_hbm.at[pl.ds(_OFFS[i], _SIZES[i]), :],
            h_buf.at[i % _NB, pl.ds(0, _SIZES[i]), :],
            in_sem.at[i % _NB])

    def out_copy(i):
        return pltpu.make_async_copy(
            o_buf.at[i % _NB, pl.ds(0, _SIZES[i]), :],
            o_hbm.at[pl.ds(_OFFS[i], _SIZES[i]), :],
            out_sem.at[i % _NB])

    for k in range(min(_NB, num)):
        in_copy(k).start()
    for i in range(num):
        in_copy(i).wait()
        hb = h_buf[i % _NB, 0:_SIZES[i], :]
        if i == 0:
            hb = _mix_first_rows(hb)
        x = jnp.dot(hb, W, preferred_element_type=jnp.float32)
        if i >= _NB:
            out_copy(i - _NB).wait()
        o_buf[i % _NB, 0:_SIZES[i], :] = jnp.tanh(x + bias)
        out_copy(i).start()
        if i + _NB < num:
            in_copy(i + _NB).start()
    for i in range(max(0, num - _NB), num):
        out_copy(i).wait()


def kernel(t, h, W, b):
    del t
    N, D = h.shape
    out = pl.pallas_call(
        _body,
        in_specs=[
            pl.BlockSpec(memory_space=pl.ANY),
            pl.BlockSpec((D, D), lambda: (0, 0)),
            pl.BlockSpec((1, D), lambda: (0, 0)),
        ],
        out_specs=pl.BlockSpec(memory_space=pl.ANY),
        out_shape=jax.ShapeDtypeStruct((N, D), jnp.float32),
        scratch_shapes=[
            pltpu.VMEM((_NB, _TMAX, D), jnp.float32),
            pltpu.VMEM((_NB, _TMAX, D), jnp.float32),
            pltpu.SemaphoreType.DMA((_NB,)),
            pltpu.SemaphoreType.DMA((_NB,)),
        ],
    )(h, W, b.reshape(1, D))
    return out


# speedup vs baseline: 1.0650x; 1.0068x over previous
import numpy as np
import jax
import jax.numpy as jnp
from jax.experimental import pallas as pl
from jax.experimental.pallas import tpu as pltpu

_CLUB = np.array([[0, 1, 1, 1],
                  [1, 0, 0, 0],
                  [1, 0, 0, 1],
                  [1, 0, 1, 0]], dtype=np.float32)
_DEG = (_CLUB + np.eye(4, dtype=np.float32)).sum(axis=0)
_DINV = 1.0 / np.sqrt(_DEG)
_A = ((_CLUB + np.eye(4, dtype=np.float32)) * _DINV[None, :] * _DINV[:, None]).astype(np.float32)

_SIZES = [1000, 3000, 8000, 16000, 16000, 16000, 16000, 16000, 5000, 3000]
_OFFS = list(np.cumsum([0] + _SIZES[:-1]))
_TMAX = max(_SIZES)
_NB = 3


def _mix_first_rows(hb):
    rows = [hb[s:s + 1, :] for s in range(4)]
    mixed = jnp.concatenate(
        [sum(float(_A[d, s]) * rows[s] for s in range(4) if _A[d, s] != 0.0)
         for d in range(4)],
        axis=0,
    )
    return jnp.concatenate([mixed, hb[4:, :]], axis=0)


def _body(h_hbm, W_ref, b_ref, o_hbm, h_buf, o_buf, in_sem, out_sem):
    W = W_ref[...]
    bias = b_ref[...]
    num = len(_SIZES)

    def in_copy(i):
        return pltpu.make_async_copy(
            h_hbm.at[pl.ds(_OFFS[i], _SIZES[i]), :],
            h_buf.at[i % _NB, pl.ds(0, _SIZES[i]), :],
            in_sem.at[i % _NB])

    def out_copy(i):
        return pltpu.make_async_copy(
            o_buf.at[i % _NB, pl.ds(0, _SIZES[i]), :],
            o_hbm.at[pl.ds(_OFFS[i], _SIZES[i]), :],
            out_sem.at[i % _NB])

    for k in range(min(_NB, num)):
        in_copy(k).start()
    for i in range(num):
        in_copy(i).wait()
        hb = h_buf[i % _NB, 0:_SIZES[i], :]
        if i == 0:
            hb = _mix_first_rows(hb)
        x = jnp.dot(hb, W, preferred_element_type=jnp.float32)
        if i >= _NB:
            out_copy(i - _NB).wait()
        o_buf[i % _NB, 0:_SIZES[i], :] = jnp.tanh(x + bias)
        out_copy(i).start()
        if i + _NB < num:
            in_copy(i + _NB).start()
    for i in range(max(0, num - _NB), num):
        out_copy(i).wait()


def kernel(t, h, W, b):
    del t
    N, D = h.shape
    out = pl.pallas_call(
        _body,
        in_specs=[
            pl.BlockSpec(memory_space=pl.ANY),
            pl.BlockSpec((D, D), lambda: (0, 0)),
            pl.BlockSpec((1, D), lambda: (0, 0)),
        ],
        out_specs=pl.BlockSpec(memory_space=pl.ANY),
        out_shape=jax.ShapeDtypeStruct((N, D), jnp.float32),
        scratch_shapes=[
            pltpu.VMEM((_NB, _TMAX, D), jnp.float32),
            pltpu.VMEM((_NB, _TMAX, D), jnp.float32),
            pltpu.SemaphoreType.DMA((_NB,)),
            pltpu.SemaphoreType.DMA((_NB,)),
        ],
    )(h, W, b.reshape(1, D))
    return out
